# Initial kernel scaffold; baseline (speedup 1.0000x reference)
import jax
import jax.numpy as jnp
from jax.experimental import pallas as pl


def kernel(u, i, j, W, H):
    raise NotImplementedError("write your pallas kernel here")


# speedup vs baseline: 1.3174x; 1.3174x over previous
import dataclasses
import functools

import jax
import jax.numpy as jnp
from jax import lax
from jax.experimental import pallas as pl
from jax.experimental.pallas import tpu as pltpu
from jax.experimental.pallas import tpu_sc as plsc

_WD = 0.025
_NW = 32
_LANES = 16
_CHUNK = 128


def _sc_gather_dot(u, i, j, W, H, batch):
    rows_per_w = batch // _NW
    n_chunks = rows_per_w // _CHUNK
    dim = W.shape[1]
    mesh = plsc.VectorSubcoreMesh(core_axis_name="c", subcore_axis_name="s")
    cp = pltpu.CompilerParams()
    if "needs_layout_passes" in pltpu.CompilerParams.__dataclass_fields__:
        cp = dataclasses.replace(cp, needs_layout_passes=False)

    @functools.partial(
        pl.kernel,
        out_type=(
            jax.ShapeDtypeStruct((batch,), jnp.float32),
            jax.ShapeDtypeStruct((_NW, _LANES), jnp.float32),
        ),
        mesh=mesh,
        compiler_params=cp,
        scratch_types=[
            pltpu.VMEM((rows_per_w,), jnp.int32),
            pltpu.VMEM((rows_per_w,), jnp.int32),
            pltpu.VMEM((rows_per_w,), jnp.int32),
            pltpu.VMEM((_CHUNK, dim), jnp.float32),
            pltpu.VMEM((_CHUNK, dim), jnp.float32),
            pltpu.VMEM((_CHUNK, dim), jnp.float32),
            pltpu.VMEM((_CHUNK,), jnp.float32),
            pltpu.VMEM((_LANES,), jnp.float32),
        ],
    )
    def sc_kernel(u_hbm, i_hbm, j_hbm, w_hbm, h_hbm, x_hbm, sq_hbm,
                  uidx, iidx, jidx, urows, irows, jrows, xout, sqv):
        wid = lax.axis_index("s") * 2 + lax.axis_index("c")
        base = wid * rows_per_w
        pltpu.sync_copy(u_hbm.at[pl.ds(base, rows_per_w)], uidx)
        pltpu.sync_copy(i_hbm.at[pl.ds(base, rows_per_w)], iidx)
        pltpu.sync_copy(j_hbm.at[pl.ds(base, rows_per_w)], jidx)
        sqv[...] = jnp.zeros((_LANES,), jnp.float32)

        @pl.loop(0, n_chunks)
        def _(c):
            off = c * _CHUNK
            pltpu.sync_copy(w_hbm.at[uidx.at[pl.ds(off, _CHUNK)]], urows)
            pltpu.sync_copy(h_hbm.at[iidx.at[pl.ds(off, _CHUNK)]], irows)
            pltpu.sync_copy(h_hbm.at[jidx.at[pl.ds(off, _CHUNK)]], jrows)

            lane = lax.iota(jnp.int32, _LANES)

            def group(g, sq):
                xvec = jnp.zeros((_LANES,), jnp.float32)
                for rr in range(_LANES):
                    r = g * _LANES + rr
                    acc = None
                    for d in range(dim // _LANES):
                        sl = pl.ds(d * _LANES, _LANES)
                        uv = urows[r, sl]
                        iv = irows[r, sl]
                        jv = jrows[r, sl]
                        px = uv * (iv - jv)
                        acc = px if acc is None else acc + px
                        sq = sq + uv * uv + iv * iv + jv * jv
                    xvec = jnp.where(lane == rr, jnp.sum(acc), xvec)
                xout[pl.ds(g * _LANES, _LANES)] = xvec
                return sq

            sq_fin = lax.fori_loop(0, _CHUNK // _LANES, group, sqv[...])
            sqv[...] = sq_fin
            pltpu.sync_copy(xout, x_hbm.at[pl.ds(base + off, _CHUNK)])

        pltpu.sync_copy(sqv, sq_hbm.at[wid])

    return sc_kernel(u, i, j, W, H)


def _tc_loss(x, sq):
    def body(x_ref, sq_ref, o_ref):
        ls = jax.nn.log_sigmoid(x_ref[...])
        reg = _WD * jnp.sum(sq_ref[...])
        o_ref[0, 0] = reg - jnp.sum(ls)

    out = pl.pallas_call(
        body,
        out_shape=jax.ShapeDtypeStruct((1, 1), jnp.float32),
        out_specs=pl.BlockSpec(memory_space=pltpu.SMEM),
    )(x.reshape(128, -1), sq)
    return out[0, 0]


def kernel(u, i, j, W, H):
    u = u.astype(jnp.int32)
    i = i.astype(jnp.int32)
    j = j.astype(jnp.int32)
    x, sq = _sc_gather_dot(u, i, j, W, H, u.shape[0])
    return _tc_loss(x, sq)


# speedup vs baseline: 1.4069x; 1.0680x over previous
import dataclasses
import functools

import jax
import jax.numpy as jnp
from jax import lax
from jax.experimental import pallas as pl
from jax.experimental.pallas import tpu as pltpu
from jax.experimental.pallas import tpu_sc as plsc

_WD = 0.025
_NW = 32
_LANES = 16
_CHUNK = 128


def _sc_gather_dot(u, i, j, W, H, batch):
    rows_per_w = batch // _NW
    n_chunks = rows_per_w // _CHUNK
    dim = W.shape[1]
    mesh = plsc.VectorSubcoreMesh(core_axis_name="c", subcore_axis_name="s")
    cp = pltpu.CompilerParams()
    if "needs_layout_passes" in pltpu.CompilerParams.__dataclass_fields__:
        cp = dataclasses.replace(cp, needs_layout_passes=False)

    @functools.partial(
        pl.kernel,
        out_type=(
            jax.ShapeDtypeStruct((batch,), jnp.float32),
            jax.ShapeDtypeStruct((_NW, _LANES), jnp.float32),
        ),
        mesh=mesh,
        compiler_params=cp,
        scratch_types=[
            pltpu.VMEM((rows_per_w,), jnp.int32),
            pltpu.VMEM((rows_per_w,), jnp.int32),
            pltpu.VMEM((rows_per_w,), jnp.int32),
            pltpu.VMEM((2, _CHUNK, dim), jnp.float32),
            pltpu.VMEM((2, _CHUNK, dim), jnp.float32),
            pltpu.VMEM((2, _CHUNK, dim), jnp.float32),
            pltpu.VMEM((_CHUNK,), jnp.float32),
            pltpu.VMEM((_LANES,), jnp.float32),
            pltpu.SemaphoreType.DMA,
            pltpu.SemaphoreType.DMA,
        ],
    )
    def sc_kernel(u_hbm, i_hbm, j_hbm, w_hbm, h_hbm, x_hbm, sq_hbm,
                  uidx, iidx, jidx, urows, irows, jrows, xout, sqv,
                  sem0, sem1):
        wid = lax.axis_index("s") * 2 + lax.axis_index("c")
        base = wid * rows_per_w
        pltpu.sync_copy(u_hbm.at[pl.ds(base, rows_per_w)], uidx)
        pltpu.sync_copy(i_hbm.at[pl.ds(base, rows_per_w)], iidx)
        pltpu.sync_copy(j_hbm.at[pl.ds(base, rows_per_w)], jidx)
        sqv[...] = jnp.zeros((_LANES,), jnp.float32)
        sems = (sem0, sem1)
        lane = lax.iota(jnp.int32, _LANES)

        def fire(c, b):
            off = c * _CHUNK
            return (
                pltpu.async_copy(
                    w_hbm.at[uidx.at[pl.ds(off, _CHUNK)]], urows.at[b], sems[b]),
                pltpu.async_copy(
                    h_hbm.at[iidx.at[pl.ds(off, _CHUNK)]], irows.at[b], sems[b]),
                pltpu.async_copy(
                    h_hbm.at[jidx.at[pl.ds(off, _CHUNK)]], jrows.at[b], sems[b]),
            )

        handles = {0: fire(0, 0)}
        for c in range(n_chunks):
            b = c % 2
            if c + 1 < n_chunks:
                handles[(c + 1) % 2] = fire(c + 1, (c + 1) % 2)
            for h in handles[b]:
                h.wait()

            def group(g, sq, b=b):
                xvec = jnp.zeros((_LANES,), jnp.float32)
                for rr in range(_LANES):
                    r = g * _LANES + rr
                    acc = None
                    for d in range(dim // _LANES):
                        sl = pl.ds(d * _LANES, _LANES)
                        uv = urows[b, r, sl]
                        iv = irows[b, r, sl]
                        jv = jrows[b, r, sl]
                        px = uv * (iv - jv)
                        acc = px if acc is None else acc + px
                        sq = sq + uv * uv + iv * iv + jv * jv
                    xvec = jnp.where(lane == rr, jnp.sum(acc), xvec)
                xout[pl.ds(g * _LANES, _LANES)] = xvec
                return sq

            sq_fin = lax.fori_loop(0, _CHUNK // _LANES, group, sqv[...])
            sqv[...] = sq_fin
            pltpu.sync_copy(xout, x_hbm.at[pl.ds(base + c * _CHUNK, _CHUNK)])

        pltpu.sync_copy(sqv, sq_hbm.at[wid])

    return sc_kernel(u, i, j, W, H)


def _tc_loss(x, sq):
    def body(x_ref, sq_ref, o_ref):
        ls = jax.nn.log_sigmoid(x_ref[...])
        reg = _WD * jnp.sum(sq_ref[...])
        o_ref[0, 0] = reg - jnp.sum(ls)

    out = pl.pallas_call(
        body,
        out_shape=jax.ShapeDtypeStruct((1, 1), jnp.float32),
        out_specs=pl.BlockSpec(memory_space=pltpu.SMEM),
    )(x.reshape(128, -1), sq)
    return out[0, 0]


def kernel(u, i, j, W, H):
    u = u.astype(jnp.int32)
    i = i.astype(jnp.int32)
    j = j.astype(jnp.int32)
    x, sq = _sc_gather_dot(u, i, j, W, H, u.shape[0])
    return _tc_loss(x, sq)


# speedup vs baseline: 1.4637x; 1.0403x over previous
import dataclasses
import functools

import jax
import jax.numpy as jnp
from jax import lax
from jax.experimental import pallas as pl
from jax.experimental.pallas import tpu as pltpu
from jax.experimental.pallas import tpu_sc as plsc

_WD = 0.025
_NW = 32
_LANES = 16
_CHUNK = 128


def _sc_gather_dot(u, i, j, W, H, batch):
    rows_per_w = batch // _NW
    n_chunks = rows_per_w // _CHUNK
    dim = W.shape[1]
    mesh = plsc.VectorSubcoreMesh(core_axis_name="c", subcore_axis_name="s")
    cp = pltpu.CompilerParams()
    if "needs_layout_passes" in pltpu.CompilerParams.__dataclass_fields__:
        cp = dataclasses.replace(cp, needs_layout_passes=False)

    @functools.partial(
        pl.kernel,
        out_type=(
            jax.ShapeDtypeStruct((batch * _LANES,), jnp.float32),
            jax.ShapeDtypeStruct((_NW, _LANES), jnp.float32),
        ),
        mesh=mesh,
        compiler_params=cp,
        scratch_types=[
            pltpu.VMEM((rows_per_w,), jnp.int32),
            pltpu.VMEM((rows_per_w,), jnp.int32),
            pltpu.VMEM((rows_per_w,), jnp.int32),
            pltpu.VMEM((2, _CHUNK, dim), jnp.float32),
            pltpu.VMEM((2, _CHUNK, dim), jnp.float32),
            pltpu.VMEM((2, _CHUNK, dim), jnp.float32),
            pltpu.VMEM((2, _CHUNK * _LANES), jnp.float32),
            pltpu.VMEM((_LANES,), jnp.float32),
            pltpu.SemaphoreType.DMA,
            pltpu.SemaphoreType.DMA,
            pltpu.SemaphoreType.DMA,
            pltpu.SemaphoreType.DMA,
        ],
    )
    def sc_kernel(u_hbm, i_hbm, j_hbm, w_hbm, h_hbm, x_hbm, sq_hbm,
                  uidx, iidx, jidx, urows, irows, jrows, xout, sqv,
                  sem0, sem1, xsem0, xsem1):
        wid = lax.axis_index("s") * 2 + lax.axis_index("c")
        base = wid * rows_per_w
        pltpu.sync_copy(u_hbm.at[pl.ds(base, rows_per_w)], uidx)
        pltpu.sync_copy(i_hbm.at[pl.ds(base, rows_per_w)], iidx)
        pltpu.sync_copy(j_hbm.at[pl.ds(base, rows_per_w)], jidx)
        sems = (sem0, sem1)
        xsems = (xsem0, xsem1)

        def fire(c, b):
            off = c * _CHUNK
            return (
                pltpu.async_copy(
                    w_hbm.at[uidx.at[pl.ds(off, _CHUNK)]], urows.at[b], sems[b]),
                pltpu.async_copy(
                    h_hbm.at[iidx.at[pl.ds(off, _CHUNK)]], irows.at[b], sems[b]),
                pltpu.async_copy(
                    h_hbm.at[jidx.at[pl.ds(off, _CHUNK)]], jrows.at[b], sems[b]),
            )

        handles = {0: fire(0, 0)}
        xhandles = {}
        sq = jnp.zeros((_LANES,), jnp.float32)
        for c in range(n_chunks):
            b = c % 2
            if c + 1 < n_chunks:
                handles[(c + 1) % 2] = fire(c + 1, (c + 1) % 2)
            for h in handles[b]:
                h.wait()
            if c >= 2:
                xhandles[b].wait()

            def body(r, sq, b=b):
                acc = su = si = sj = None
                for d in range(dim // _LANES):
                    sl = pl.ds(d * _LANES, _LANES)
                    uv = urows[b, r, sl]
                    iv = irows[b, r, sl]
                    jv = jrows[b, r, sl]
                    px = uv * (iv - jv)
                    if d == 0:
                        acc, su, si, sj = px, uv * uv, iv * iv, jv * jv
                    else:
                        acc = acc + px
                        su = su + uv * uv
                        si = si + iv * iv
                        sj = sj + jv * jv
                xout[b, pl.ds(r * _LANES, _LANES)] = acc
                return sq + ((su + si) + sj)

            sq = plsc.parallel_loop(0, _CHUNK, unroll=4, carry=sq)(body)
            xhandles[b] = pltpu.async_copy(
                xout.at[b],
                x_hbm.at[pl.ds((base + c * _CHUNK) * _LANES, _CHUNK * _LANES)],
                xsems[b])

        sqv[...] = sq
        for b in sorted(xhandles):
            xhandles[b].wait()
        pltpu.sync_copy(sqv, sq_hbm.at[wid])

    return sc_kernel(u, i, j, W, H)


def _tc_loss(xpart, sq):
    def body(x_ref, sq_ref, o_ref):
        x2 = x_ref[...]
        lmap = lax.broadcasted_iota(jnp.int32, (128, 8), 0)
        gmap = lax.broadcasted_iota(jnp.int32, (128, 8), 1)
        sel = (lmap // _LANES == gmap).astype(jnp.float32)
        logits = lax.dot_general(
            x2, sel, (((1,), (0,)), ((), ())),
            preferred_element_type=jnp.float32,
            precision=lax.Precision.HIGHEST)
        ls = jax.nn.log_sigmoid(logits)
        reg = _WD * jnp.sum(sq_ref[...])
        o_ref[0, 0] = reg - jnp.sum(ls)

    out = pl.pallas_call(
        body,
        out_shape=jax.ShapeDtypeStruct((1, 1), jnp.float32),
        out_specs=pl.BlockSpec(memory_space=pltpu.SMEM),
    )(xpart.reshape(-1, 128), sq)
    return out[0, 0]


def kernel(u, i, j, W, H):
    u = u.astype(jnp.int32)
    i = i.astype(jnp.int32)
    j = j.astype(jnp.int32)
    x, sq = _sc_gather_dot(u, i, j, W, H, u.shape[0])
    return _tc_loss(x, sq)


# speedup vs baseline: 1.8739x; 1.2803x over previous
import dataclasses
import functools

import jax
import jax.numpy as jnp
from jax import lax
from jax.experimental import pallas as pl
from jax.experimental.pallas import tpu as pltpu
from jax.experimental.pallas import tpu_sc as plsc

_WD = 0.025
_NW = 32
_LANES = 16
_CHUNK = 128


def _sc_gather_dot(u, i, j, W, H, batch):
    rows_per_w = batch // _NW
    n_chunks = rows_per_w // _CHUNK
    dim = W.shape[1]
    mesh = plsc.VectorSubcoreMesh(core_axis_name="c", subcore_axis_name="s")
    cp = pltpu.CompilerParams()
    if "needs_layout_passes" in pltpu.CompilerParams.__dataclass_fields__:
        cp = dataclasses.replace(cp, needs_layout_passes=False)

    @functools.partial(
        pl.kernel,
        out_type=jax.ShapeDtypeStruct((batch * 2 * _LANES,), jnp.float32),
        mesh=mesh,
        compiler_params=cp,
        scratch_types=[
            pltpu.VMEM((rows_per_w,), jnp.int32),
            pltpu.VMEM((rows_per_w,), jnp.int32),
            pltpu.VMEM((rows_per_w,), jnp.int32),
            pltpu.VMEM((2, _CHUNK, dim), jnp.float32),
            pltpu.VMEM((2, _CHUNK, dim), jnp.float32),
            pltpu.VMEM((2, _CHUNK, dim), jnp.float32),
            pltpu.VMEM((2, _CHUNK * 2 * _LANES), jnp.float32),
            pltpu.SemaphoreType.DMA,
            pltpu.SemaphoreType.DMA,
            pltpu.SemaphoreType.DMA,
            pltpu.SemaphoreType.DMA,
        ],
    )
    def sc_kernel(u_hbm, i_hbm, j_hbm, w_hbm, h_hbm, x_hbm,
                  uidx, iidx, jidx, urows, irows, jrows, xout,
                  sem0, sem1, xsem0, xsem1):
        wid = lax.axis_index("s") * 2 + lax.axis_index("c")
        base = wid * rows_per_w
        pltpu.sync_copy(u_hbm.at[pl.ds(base, rows_per_w)], uidx)
        pltpu.sync_copy(i_hbm.at[pl.ds(base, rows_per_w)], iidx)
        pltpu.sync_copy(j_hbm.at[pl.ds(base, rows_per_w)], jidx)
        sems = (sem0, sem1)
        xsems = (xsem0, xsem1)

        def fire(c, b):
            off = c * _CHUNK
            return (
                pltpu.async_copy(
                    w_hbm.at[uidx.at[pl.ds(off, _CHUNK)]], urows.at[b], sems[b]),
                pltpu.async_copy(
                    h_hbm.at[iidx.at[pl.ds(off, _CHUNK)]], irows.at[b], sems[b]),
                pltpu.async_copy(
                    h_hbm.at[jidx.at[pl.ds(off, _CHUNK)]], jrows.at[b], sems[b]),
            )

        handles = {0: fire(0, 0)}
        xhandles = {}
        for c in range(n_chunks):
            b = c % 2
            if c + 1 < n_chunks:
                handles[(c + 1) % 2] = fire(c + 1, (c + 1) % 2)
            for h in handles[b]:
                h.wait()
            if c >= 2:
                xhandles[b].wait()

            @plsc.parallel_loop(0, _CHUNK, unroll=8)
            def _(r, b=b):
                acc = su = si = sj = None
                for d in range(dim // _LANES):
                    sl = pl.ds(d * _LANES, _LANES)
                    uv = urows[b, r, sl]
                    iv = irows[b, r, sl]
                    jv = jrows[b, r, sl]
                    px = uv * (iv - jv)
                    if d == 0:
                        acc, su, si, sj = px, uv * uv, iv * iv, jv * jv
                    else:
                        acc = acc + px
                        su = su + uv * uv
                        si = si + iv * iv
                        sj = sj + jv * jv
                xout[b, pl.ds(r * 2 * _LANES, _LANES)] = acc
                xout[b, pl.ds(r * 2 * _LANES + _LANES, _LANES)] = (su + si) + sj

            xhandles[b] = pltpu.async_copy(
                xout.at[b],
                x_hbm.at[pl.ds((base + c * _CHUNK) * 2 * _LANES,
                               _CHUNK * 2 * _LANES)],
                xsems[b])

        for b in sorted(xhandles):
            xhandles[b].wait()

    return sc_kernel(u, i, j, W, H)


def _tc_loss(xpart):
    def body(x_ref, o_ref):
        x2 = x_ref[...]
        lmap = lax.broadcasted_iota(jnp.int32, (128, 4), 0)
        gmap = lax.broadcasted_iota(jnp.int32, (128, 4), 1)
        sel = ((lmap // (2 * _LANES) == gmap)
               & (lmap % (2 * _LANES) < _LANES)).astype(jnp.float32)
        logits = lax.dot_general(
            x2, sel, (((1,), (0,)), ((), ())),
            preferred_element_type=jnp.float32,
            precision=lax.Precision.HIGHEST)
        ls = jax.nn.log_sigmoid(logits)
        sum_logits = jnp.sum(logits)
        reg = _WD * (jnp.sum(x2) - sum_logits)
        o_ref[0, 0] = reg - jnp.sum(ls)

    out = pl.pallas_call(
        body,
        out_shape=jax.ShapeDtypeStruct((1, 1), jnp.float32),
        out_specs=pl.BlockSpec(memory_space=pltpu.SMEM),
    )(xpart.reshape(-1, 128))
    return out[0, 0]


def kernel(u, i, j, W, H):
    u = u.astype(jnp.int32)
    i = i.astype(jnp.int32)
    j = j.astype(jnp.int32)
    x = _sc_gather_dot(u, i, j, W, H, u.shape[0])
    return _tc_loss(x)
